# baseline (speedup 1.0000x reference)
import functools

import jax
import jax.numpy as jnp
from jax import lax
from jax.experimental import pallas as pl
from jax.experimental.pallas import tpu as pltpu
from jax.experimental.pallas import tpu_sc as plsc

B = 4096
L = 200
D_IN = 64
STATION_DIM = 16
SEASON_DIM = 4
N_SEASONS = 4
D_OUT = D_IN + STATION_DIM + SEASON_DIM

_NC = 2
_NS = 16
_NW = _NC * _NS
_B_PER_W = B // _NW


def _sc_assemble(x, station_ids, season_ids, station_table, season_table):
    mesh = plsc.VectorSubcoreMesh(core_axis_name="c", subcore_axis_name="s")

    @functools.partial(
        pl.kernel,
        mesh=mesh,
        out_type=jax.ShapeDtypeStruct((B, L, D_OUT), jnp.float32),
        scratch_types=[
            pltpu.VMEM((_B_PER_W,), jnp.int32),
            pltpu.VMEM((_B_PER_W, STATION_DIM), jnp.float32),
            pltpu.VMEM((2, L, STATION_DIM), jnp.float32),
            pltpu.VMEM((2, L, SEASON_DIM), jnp.float32),
            pltpu.VMEM((N_SEASONS, SEASON_DIM), jnp.float32),
            pltpu.VMEM((_B_PER_W,), jnp.int32),
            pltpu.SemaphoreType.DMA,
            pltpu.SemaphoreType.DMA,
            pltpu.SemaphoreType.DMA,
            pltpu.SemaphoreType.DMA,
        ],
        compiler_params=pltpu.CompilerParams(use_tc_tiling_on_sc=False,
                                             needs_layout_passes=False),
    )
    def k(x_hbm, sid_hbm, seid_hbm, table_hbm, stab_hbm, out_hbm,
          idx_v, st_rows, st_bc, se_bc, stab_v, sed_v,
          sem, sem_x, sem_st, sem_se):
        wid = lax.axis_index("s") * _NC + lax.axis_index("c")
        base = wid * _B_PER_W
        pltpu.sync_copy(sid_hbm.at[pl.ds(base, _B_PER_W)], idx_v)
        pltpu.async_copy(table_hbm.at[idx_v], st_rows, sem).wait()
        pltpu.sync_copy(seid_hbm.at[pl.ds(base, _B_PER_W)], sed_v)
        pltpu.sync_copy(stab_hbm, stab_v)

        lanes = lax.iota(jnp.int32, 16)
        _XLAG = 8

        def body(j, carry):
            b = base + j
            p = j & 1
            pltpu.make_async_copy(x_hbm.at[b], out_hbm.at[b, :, 0:D_IN],
                                  sem_x).start()
            @pl.when(j >= _XLAG)
            def _():
                pltpu.make_async_copy(
                    x_hbm.at[b], out_hbm.at[b, :, 0:D_IN], sem_x).wait()
            @pl.when(j >= 2)
            def _():
                bp = base + j - 2
                pltpu.make_async_copy(
                    st_bc.at[p],
                    out_hbm.at[bp, :, D_IN:D_IN + STATION_DIM],
                    sem_st).wait()
                pltpu.make_async_copy(
                    se_bc.at[p],
                    out_hbm.at[bp, :, D_IN + STATION_DIM:D_OUT],
                    sem_se).wait()
            stv = st_rows[j, :]
            def fill_l(l, c0):
                st_bc[p, l, :] = stv
                return c0
            lax.fori_loop(0, L, fill_l, 0)
            sid_splat = plsc.load_gather(
                sed_v, [jnp.full((16,), j, jnp.int32)])
            pat = plsc.load_gather(stab_v, [sid_splat, lanes & 3])
            def fill_t(t, c0):
                flat = t * 16 + lanes
                plsc.store_scatter(
                    se_bc, [jnp.full((16,), p, jnp.int32), flat >> 2,
                            flat & 3], pat)
                return c0
            lax.fori_loop(0, L * SEASON_DIM // 16, fill_t, 0)
            pltpu.make_async_copy(
                st_bc.at[p], out_hbm.at[b, :, D_IN:D_IN + STATION_DIM],
                sem_st).start()
            pltpu.make_async_copy(
                se_bc.at[p], out_hbm.at[b, :, D_IN + STATION_DIM:D_OUT],
                sem_se).start()
            return carry
        lax.fori_loop(0, _B_PER_W, body, 0)

        def drain_x(j, carry):
            b = base + j
            pltpu.make_async_copy(
                x_hbm.at[b], out_hbm.at[b, :, 0:D_IN], sem_x).wait()
            return carry
        lax.fori_loop(_B_PER_W - _XLAG, _B_PER_W, drain_x, 0)

        def drain_bc(j, carry):
            b = base + j
            pltpu.make_async_copy(
                st_bc.at[j & 1],
                out_hbm.at[b, :, D_IN:D_IN + STATION_DIM], sem_st).wait()
            pltpu.make_async_copy(
                se_bc.at[j & 1],
                out_hbm.at[b, :, D_IN + STATION_DIM:D_OUT], sem_se).wait()
            return carry
        lax.fori_loop(_B_PER_W - 2, _B_PER_W, drain_bc, 0)

    return k(x, station_ids, season_ids, station_table, season_table)


def kernel(x, station_ids, season_ids, station_table, season_table):
    return _sc_assemble(x, station_ids, season_ids, station_table,
                        season_table)


# speedup vs baseline: 6.0592x; 6.0592x over previous
import functools

import jax
import jax.numpy as jnp
from jax import lax
from jax.experimental import pallas as pl
from jax.experimental.pallas import tpu as pltpu
from jax.experimental.pallas import tpu_sc as plsc

B = 4096
L = 200
D_IN = 64
STATION_DIM = 16
SEASON_DIM = 4
N_SEASONS = 4
D_OUT = D_IN + STATION_DIM + SEASON_DIM

_NC = 2
_NS = 16
_NW = _NC * _NS
_B_PER_W = B // _NW


def _sc_assemble(x, station_ids, season_ids, station_table, season_table):
    mesh = plsc.VectorSubcoreMesh(core_axis_name="c", subcore_axis_name="s")

    @functools.partial(
        pl.kernel,
        mesh=mesh,
        out_type=jax.ShapeDtypeStruct((B, L, D_OUT), jnp.float32),
        scratch_types=[
            pltpu.VMEM((_B_PER_W,), jnp.int32),
            pltpu.VMEM((_B_PER_W, STATION_DIM), jnp.float32),
            pltpu.VMEM((4, L, D_IN), jnp.float32),
            pltpu.VMEM((2, L, STATION_DIM), jnp.float32),
            pltpu.VMEM((2, L, SEASON_DIM), jnp.float32),
            pltpu.VMEM((N_SEASONS, SEASON_DIM), jnp.float32),
            pltpu.VMEM((_B_PER_W,), jnp.int32),
            pltpu.SemaphoreType.DMA,
            pltpu.SemaphoreType.DMA,
            pltpu.SemaphoreType.DMA,
            pltpu.SemaphoreType.DMA,
            pltpu.SemaphoreType.DMA,
        ],
        compiler_params=pltpu.CompilerParams(use_tc_tiling_on_sc=False,
                                             needs_layout_passes=False),
    )
    def k(x_hbm, sid_hbm, seid_hbm, table_hbm, stab_hbm, out_hbm,
          idx_v, st_rows, xbuf, st_bc, se_bc, stab_v, sed_v,
          sem, sem_xi, sem_xo, sem_st, sem_se):
        wid = lax.axis_index("s") * _NC + lax.axis_index("c")
        base = wid * _B_PER_W
        pltpu.sync_copy(sid_hbm.at[pl.ds(base, _B_PER_W)], idx_v)
        pltpu.async_copy(table_hbm.at[idx_v], st_rows, sem).wait()
        pltpu.sync_copy(seid_hbm.at[pl.ds(base, _B_PER_W)], sed_v)
        pltpu.sync_copy(stab_hbm, stab_v)

        lanes = lax.iota(jnp.int32, 16)

        pltpu.make_async_copy(x_hbm.at[base], xbuf.at[0], sem_xi).start()
        pltpu.make_async_copy(x_hbm.at[base + 1], xbuf.at[1], sem_xi).start()

        def body(j, carry):
            b = base + j
            p = j & 1
            q = j % 4
            pltpu.make_async_copy(x_hbm.at[b], xbuf.at[q], sem_xi).wait()
            pltpu.make_async_copy(xbuf.at[q], out_hbm.at[b, :, 0:D_IN],
                                  sem_xo).start()
            @pl.when(j >= 2)
            def _():
                bp = base + j - 2
                pltpu.make_async_copy(
                    st_bc.at[p],
                    out_hbm.at[bp, :, D_IN:D_IN + STATION_DIM],
                    sem_st).wait()
                pltpu.make_async_copy(
                    se_bc.at[p],
                    out_hbm.at[bp, :, D_IN + STATION_DIM:D_OUT],
                    sem_se).wait()
            stv = st_rows[j, :]
            def fill_l(l, c0):
                st_bc[p, l, :] = stv
                return c0
            lax.fori_loop(0, L, fill_l, 0)
            sid_splat = plsc.load_gather(
                sed_v, [jnp.full((16,), j, jnp.int32)])
            pat = plsc.load_gather(stab_v, [sid_splat, lanes & 3])
            def fill_t(t, c0):
                flat = t * 16 + lanes
                plsc.store_scatter(
                    se_bc, [jnp.full((16,), p, jnp.int32), flat >> 2,
                            flat & 3], pat)
                return c0
            lax.fori_loop(0, L * SEASON_DIM // 16, fill_t, 0)
            pltpu.make_async_copy(
                st_bc.at[p], out_hbm.at[b, :, D_IN:D_IN + STATION_DIM],
                sem_st).start()
            pltpu.make_async_copy(
                se_bc.at[p], out_hbm.at[b, :, D_IN + STATION_DIM:D_OUT],
                sem_se).start()
            @pl.when(j >= 1)
            def _():
                pltpu.make_async_copy(
                    xbuf.at[q], out_hbm.at[b, :, 0:D_IN], sem_xo).wait()
            @pl.when(j + 2 < _B_PER_W)
            def _():
                pltpu.make_async_copy(
                    x_hbm.at[b + 2], xbuf.at[(j + 2) % 4], sem_xi).start()
            return carry
        lax.fori_loop(0, _B_PER_W, body, 0)

        pltpu.make_async_copy(
            xbuf.at[0], out_hbm.at[base, :, 0:D_IN], sem_xo).wait()

        def drain_bc(j, carry):
            b = base + j
            pltpu.make_async_copy(
                st_bc.at[j & 1],
                out_hbm.at[b, :, D_IN:D_IN + STATION_DIM], sem_st).wait()
            pltpu.make_async_copy(
                se_bc.at[j & 1],
                out_hbm.at[b, :, D_IN + STATION_DIM:D_OUT], sem_se).wait()
            return carry
        lax.fori_loop(_B_PER_W - 2, _B_PER_W, drain_bc, 0)

    return k(x, station_ids, season_ids, station_table, season_table)


def kernel(x, station_ids, season_ids, station_table, season_table):
    return _sc_assemble(x, station_ids, season_ids, station_table,
                        season_table)


# speedup vs baseline: 6.0603x; 1.0002x over previous
import functools

import jax
import jax.numpy as jnp
from jax import lax
from jax.experimental import pallas as pl
from jax.experimental.pallas import tpu as pltpu
from jax.experimental.pallas import tpu_sc as plsc

B = 4096
L = 200
D_IN = 64
STATION_DIM = 16
SEASON_DIM = 4
N_SEASONS = 4
D_OUT = D_IN + STATION_DIM + SEASON_DIM

_NC = 2
_NS = 16
_NW = _NC * _NS
_B_PER_W = B // _NW
_G = 2
_NSTEP = _B_PER_W // _G


def _sc_assemble(x, station_ids, season_ids, station_table, season_table):
    mesh = plsc.VectorSubcoreMesh(core_axis_name="c", subcore_axis_name="s")

    @functools.partial(
        pl.kernel,
        mesh=mesh,
        out_type=jax.ShapeDtypeStruct((B, L, D_OUT), jnp.float32),
        scratch_types=[
            pltpu.VMEM((_B_PER_W,), jnp.int32),
            pltpu.VMEM((_B_PER_W, STATION_DIM), jnp.float32),
            pltpu.VMEM((3, _G, L, D_IN), jnp.float32),
            pltpu.VMEM((2, _G, L, STATION_DIM), jnp.float32),
            pltpu.VMEM((2, _G, L, SEASON_DIM), jnp.float32),
            pltpu.VMEM((N_SEASONS, SEASON_DIM), jnp.float32),
            pltpu.VMEM((_B_PER_W,), jnp.int32),
            pltpu.SemaphoreType.DMA,
            pltpu.SemaphoreType.DMA,
            pltpu.SemaphoreType.DMA,
            pltpu.SemaphoreType.DMA,
            pltpu.SemaphoreType.DMA,
        ],
        compiler_params=pltpu.CompilerParams(use_tc_tiling_on_sc=False,
                                             needs_layout_passes=False),
    )
    def k(x_hbm, sid_hbm, seid_hbm, table_hbm, stab_hbm, out_hbm,
          idx_v, st_rows, xbuf, st_bc, se_bc, stab_v, sed_v,
          sem, sem_xi, sem_xo, sem_st, sem_se):
        wid = lax.axis_index("s") * _NC + lax.axis_index("c")
        base = wid * _B_PER_W
        pltpu.sync_copy(sid_hbm.at[pl.ds(base, _B_PER_W)], idx_v)
        pltpu.async_copy(table_hbm.at[idx_v], st_rows, sem).wait()
        pltpu.sync_copy(seid_hbm.at[pl.ds(base, _B_PER_W)], sed_v)
        pltpu.sync_copy(stab_hbm, stab_v)

        lanes = lax.iota(jnp.int32, 16)

        pltpu.make_async_copy(x_hbm.at[pl.ds(base, _G)], xbuf.at[0],
                              sem_xi).start()

        def body(m, carry):
            b = base + m * _G
            pp = m & 1
            q = m % 3
            pltpu.make_async_copy(x_hbm.at[pl.ds(b, _G)], xbuf.at[q],
                                  sem_xi).wait()
            pltpu.make_async_copy(xbuf.at[q],
                                  out_hbm.at[pl.ds(b, _G), :, 0:D_IN],
                                  sem_xo).start()
            @pl.when(m >= 2)
            def _():
                pltpu.make_async_copy(
                    st_bc.at[pp],
                    out_hbm.at[pl.ds(b, _G), :, D_IN:D_IN + STATION_DIM],
                    sem_st).wait()
                pltpu.make_async_copy(
                    se_bc.at[pp],
                    out_hbm.at[pl.ds(b, _G), :, D_IN + STATION_DIM:D_OUT],
                    sem_se).wait()
            for r in range(_G):
                j = m * _G + r
                stv = st_rows[j, :]
                def fill_l(mm, c0, _r=r, _stv=stv):
                    ll = mm * 8
                    for i in range(8):
                        st_bc[pp, _r, ll + i, :] = _stv
                    return c0
                lax.fori_loop(0, L // 8, fill_l, 0)
                sid_splat = plsc.load_gather(
                    sed_v, [jnp.full((16,), j, jnp.int32)])
                pat = plsc.load_gather(stab_v, [sid_splat, lanes & 3])
                def fill_t(tt, c0, _r=r, _pat=pat):
                    for i in range(10):
                        flat = (tt * 10 + i) * 16 + lanes
                        plsc.store_scatter(
                            se_bc,
                            [jnp.full((16,), pp, jnp.int32),
                             jnp.full((16,), _r, jnp.int32),
                             flat >> 2, flat & 3], _pat)
                    return c0
                lax.fori_loop(0, L * SEASON_DIM // 160, fill_t, 0)
            pltpu.make_async_copy(
                st_bc.at[pp],
                out_hbm.at[pl.ds(b, _G), :, D_IN:D_IN + STATION_DIM],
                sem_st).start()
            pltpu.make_async_copy(
                se_bc.at[pp],
                out_hbm.at[pl.ds(b, _G), :, D_IN + STATION_DIM:D_OUT],
                sem_se).start()
            @pl.when(m >= 2)
            def _():
                pltpu.make_async_copy(
                    xbuf.at[q], out_hbm.at[pl.ds(b, _G), :, 0:D_IN],
                    sem_xo).wait()
            @pl.when(m + 1 < _NSTEP)
            def _():
                pltpu.make_async_copy(
                    x_hbm.at[pl.ds(b + _G, _G)], xbuf.at[(m + 1) % 3],
                    sem_xi).start()
            return carry
        lax.fori_loop(0, _NSTEP, body, 0)

        def drain(i, carry):
            pltpu.make_async_copy(
                xbuf.at[0], out_hbm.at[pl.ds(base, _G), :, 0:D_IN],
                sem_xo).wait()
            pltpu.make_async_copy(
                st_bc.at[0],
                out_hbm.at[pl.ds(base, _G), :, D_IN:D_IN + STATION_DIM],
                sem_st).wait()
            pltpu.make_async_copy(
                se_bc.at[0],
                out_hbm.at[pl.ds(base, _G), :, D_IN + STATION_DIM:D_OUT],
                sem_se).wait()
            return carry
        lax.fori_loop(0, 2, drain, 0)

    return k(x, station_ids, season_ids, station_table, season_table)


def kernel(x, station_ids, season_ids, station_table, season_table):
    return _sc_assemble(x, station_ids, season_ids, station_table,
                        season_table)
